# baseline (speedup 1.0000x reference)
import functools

import jax
import jax.numpy as jnp
from jax import lax
from jax.experimental import pallas as pl
from jax.experimental.pallas import tpu as pltpu
from jax.experimental.pallas import tpu_sc as plsc

CODEBOOK_SIZE = 1024
CODE_DIM = 256
COMMITMENT_WEIGHT = 0.25

TOK_BLK = 512
NBLK = 9
N_TOK = NBLK * TOK_BLK

NSLICE = 3
SLICE_BLKS = NBLK // NSLICE
SLICE_TOK = SLICE_BLKS * TOK_BLK

NC, NS = 2, 16
NW = NC * NS
ROWS_PER_W = SLICE_TOK // NW


def _dist_argmin_body(z_ref, cb_ref, idx_ref, loss_ref, cbsq_ref):
    i = pl.program_id(0)
    z = z_ref[0]
    cb = cb_ref[...]

    @pl.when(i == 0)
    def _prep():
        cbsq_ref[...] = jnp.sum(cb * cb, axis=1, keepdims=True)

    scores_t = lax.dot_general(
        cb, z, (((1,), (1,)), ((), ())),
        preferred_element_type=jnp.float32)
    dist_t = cbsq_ref[...] - 2.0 * scores_t
    min_val = jnp.min(dist_t, axis=0, keepdims=True)
    row = lax.broadcasted_iota(jnp.int32, dist_t.shape, 0)
    idx = jnp.min(jnp.where(dist_t == min_val, row, jnp.int32(CODEBOOK_SIZE)),
                  axis=0, keepdims=True)
    idx_ref[0] = idx

    @pl.when(i == 0)
    def _init():
        loss_ref[0, 0] = 0.0

    loss_ref[0, 0] += jnp.sum(min_val) + jnp.sum(z * z)

    @pl.when(i == SLICE_BLKS - 1)
    def _scale():
        total = jnp.float32(N_TOK * CODE_DIM)
        loss_ref[0, 0] = loss_ref[0, 0] * (
            (1.0 + COMMITMENT_WEIGHT) / total)


_dist_argmin = pl.pallas_call(
    _dist_argmin_body,
    grid=(SLICE_BLKS,),
    in_specs=[
        pl.BlockSpec((1, TOK_BLK, CODE_DIM), lambda i: (i, 0, 0)),
        pl.BlockSpec((CODEBOOK_SIZE, CODE_DIM), lambda i: (0, 0)),
    ],
    out_specs=[
        pl.BlockSpec((1, 1, TOK_BLK), lambda i: (i, 0, 0)),
        pl.BlockSpec(memory_space=pltpu.SMEM),
    ],
    out_shape=[
        jax.ShapeDtypeStruct((SLICE_BLKS, 1, TOK_BLK), jnp.int32),
        jax.ShapeDtypeStruct((1, 1), jnp.float32),
    ],
    scratch_shapes=[
        pltpu.VMEM((CODEBOOK_SIZE, 1), jnp.float32),
    ],
)


@functools.cache
def _make_sc_gather():
    mesh = plsc.VectorSubcoreMesh(core_axis_name="c", subcore_axis_name="s")

    @functools.partial(
        pl.kernel,
        mesh=mesh,
        out_type=jax.ShapeDtypeStruct((SLICE_TOK, CODE_DIM), jnp.float32),
        scratch_types=[
            pltpu.VMEM((ROWS_PER_W,), jnp.int32),
            pltpu.VMEM((ROWS_PER_W, CODE_DIM), jnp.float32),
            pltpu.SemaphoreType.DMA,
        ],
    )
    def _sc_gather(cb_hbm, idx_hbm, out_hbm, idx_v, rows_v, sem):
        wid = lax.axis_index("s") * NC + lax.axis_index("c")
        base = wid * ROWS_PER_W
        pltpu.sync_copy(idx_hbm.at[pl.ds(base, ROWS_PER_W)], idx_v)
        pltpu.async_copy(cb_hbm.at[idx_v], rows_v, sem).wait()
        pltpu.sync_copy(rows_v, out_hbm.at[pl.ds(base, ROWS_PER_W)])

    return _sc_gather


def kernel(z, codebook):
    B, N, D = z.shape
    z_slices = z.reshape(NSLICE, SLICE_BLKS, TOK_BLK, D)
    sc_gather = _make_sc_gather()
    idx_parts = []
    q_parts = []
    loss_parts = []
    for s in range(NSLICE):
        idx3, loss_acc = _dist_argmin(z_slices[s], codebook)
        idx_flat = idx3.reshape(-1)
        q_parts.append(sc_gather(codebook, idx_flat))
        idx_parts.append(idx_flat)
        loss_parts.append(loss_acc[0, 0])
    quantized_st = jnp.stack(q_parts).reshape(B, N, D)
    indices = jnp.stack(idx_parts).reshape(B, N)
    loss = loss_parts[0] + loss_parts[1] + loss_parts[2]
    return quantized_st, indices, loss


# speedup vs baseline: 1.4859x; 1.4859x over previous
import functools

import jax
import jax.numpy as jnp
from jax import lax
from jax.experimental import pallas as pl
from jax.experimental.pallas import tpu as pltpu
from jax.experimental.pallas import tpu_sc as plsc

CODEBOOK_SIZE = 1024
CODE_DIM = 256
COMMITMENT_WEIGHT = 0.25

TOK_BLK = 768
HALF_BLKS = 3
HALF_TOK = HALF_BLKS * TOK_BLK
N_TOK = 2 * HALF_TOK

NC, NS = 2, 16
NW = NC * NS
ROWS_PER_W = HALF_TOK // NW


def _argmin_half(z_ref, cb_ref, idx_ref, loss_ref, cbsq_ref, *, gather):
    i = pl.program_id(0)
    z = z_ref[0]
    cb = cb_ref[...]

    @pl.when(i == 0)
    def _prep():
        cbsq_ref[...] = jnp.sum(cb * cb, axis=1, keepdims=True)

    scores_t = lax.dot_general(
        cb, z, (((1,), (1,)), ((), ())),
        preferred_element_type=jnp.float32)
    dist_t = cbsq_ref[...] - 2.0 * scores_t
    min_val = jnp.min(dist_t, axis=0, keepdims=True)
    row = lax.broadcasted_iota(jnp.int32, dist_t.shape, 0)
    idx = jnp.min(jnp.where(dist_t == min_val, row, jnp.int32(CODEBOOK_SIZE)),
                  axis=0, keepdims=True)
    idx_ref[pl.ds(i * TOK_BLK, TOK_BLK)] = idx[0]

    @pl.when(i == 0)
    def _init():
        loss_ref[0, 0] = 0.0

    loss_ref[0, 0] += jnp.sum(min_val) + jnp.sum(z * z)
    return dist_t, row, idx


def _dist_argmin_a_body(z_ref, cb_ref, idx_ref, loss_ref, cbsq_ref):
    _argmin_half(z_ref, cb_ref, idx_ref, loss_ref, cbsq_ref, gather=False)


def _dist_argmin_b_body(z_ref, cb_ref, loss_a_ref, idx_ref, loss_ref, q_ref,
                        cbsq_ref):
    i = pl.program_id(0)
    dist_t, row, idx = _argmin_half(
        z_ref, cb_ref, idx_ref, loss_ref, cbsq_ref, gather=True)

    @pl.when(i == 0)
    def _carry():
        loss_ref[0, 0] += loss_a_ref[0, 0]

    onehot_t = jnp.where(row == idx, 1.0, 0.0)
    q_ref[...] = lax.dot_general(
        onehot_t, cb_ref[...], (((0,), (0,)), ((), ())),
        preferred_element_type=jnp.float32)

    @pl.when(i == HALF_BLKS - 1)
    def _scale():
        total = jnp.float32(N_TOK * CODE_DIM)
        loss_ref[0, 0] = loss_ref[0, 0] * (
            (1.0 + COMMITMENT_WEIGHT) / total)


_common = dict(
    grid=(HALF_BLKS,),
    scratch_shapes=[pltpu.VMEM((CODEBOOK_SIZE, 1), jnp.float32)],
)

_dist_argmin_a = pl.pallas_call(
    _dist_argmin_a_body,
    in_specs=[
        pl.BlockSpec((1, TOK_BLK, CODE_DIM), lambda i: (i, 0, 0)),
        pl.BlockSpec((CODEBOOK_SIZE, CODE_DIM), lambda i: (0, 0)),
    ],
    out_specs=[
        pl.BlockSpec((HALF_TOK,), lambda i: (0,)),
        pl.BlockSpec(memory_space=pltpu.SMEM),
    ],
    out_shape=[
        jax.ShapeDtypeStruct((HALF_TOK,), jnp.int32),
        jax.ShapeDtypeStruct((1, 1), jnp.float32),
    ],
    **_common,
)

_B_OFF = HALF_BLKS

_dist_argmin_b = pl.pallas_call(
    _dist_argmin_b_body,
    in_specs=[
        pl.BlockSpec((1, TOK_BLK, CODE_DIM), lambda i: (i + _B_OFF, 0, 0)),
        pl.BlockSpec((CODEBOOK_SIZE, CODE_DIM), lambda i: (0, 0)),
        pl.BlockSpec(memory_space=pltpu.SMEM),
    ],
    out_specs=[
        pl.BlockSpec((HALF_TOK,), lambda i: (0,)),
        pl.BlockSpec(memory_space=pltpu.SMEM),
        pl.BlockSpec((TOK_BLK, CODE_DIM), lambda i: (i, 0)),
    ],
    out_shape=[
        jax.ShapeDtypeStruct((HALF_TOK,), jnp.int32),
        jax.ShapeDtypeStruct((1, 1), jnp.float32),
        jax.ShapeDtypeStruct((HALF_TOK, CODE_DIM), jnp.float32),
    ],
    **_common,
)


@functools.cache
def _make_sc_gather():
    mesh = plsc.VectorSubcoreMesh(core_axis_name="c", subcore_axis_name="s")

    @functools.partial(
        pl.kernel,
        mesh=mesh,
        out_type=jax.ShapeDtypeStruct((N_TOK, CODE_DIM), jnp.float32),
        scratch_types=[
            pltpu.VMEM((ROWS_PER_W,), jnp.int32),
            pltpu.VMEM((ROWS_PER_W, CODE_DIM), jnp.float32),
            pltpu.SemaphoreType.DMA,
        ],
    )
    def _sc_gather(cb_hbm, idx_hbm, out_hbm, idx_v, rows_v, sem):
        wid = lax.axis_index("s") * NC + lax.axis_index("c")
        base = wid * ROWS_PER_W
        pltpu.sync_copy(idx_hbm.at[pl.ds(base, ROWS_PER_W)], idx_v)
        pltpu.async_copy(cb_hbm.at[idx_v], rows_v, sem).wait()
        pltpu.sync_copy(rows_v, out_hbm.at[pl.ds(base, ROWS_PER_W)])

    return _sc_gather


def kernel(z, codebook):
    B, N, D = z.shape
    z_blocks = z.reshape(2 * HALF_BLKS, TOK_BLK, D)
    idx_a, loss_a = _dist_argmin_a(z_blocks, codebook)
    q_full = _make_sc_gather()(codebook, idx_a)
    idx_b, loss_b, q_b = _dist_argmin_b(z_blocks, codebook, loss_a)
    q_full = lax.dynamic_update_slice(q_full, q_b, (HALF_TOK, 0))
    quantized_st = q_full.reshape(B, N, D)
    indices = jnp.concatenate([idx_a, idx_b]).reshape(B, N)
    loss = loss_b[0, 0]
    return quantized_st, indices, loss
